# baseline (speedup 1.0000x reference)
import functools

import jax
import jax.numpy as jnp
from jax import lax
from jax.experimental import pallas as pl
from jax.experimental.pallas import tpu as pltpu
from jax.experimental.pallas import tpu_sc as plsc

N_NODES = 100000
D = 128
B = 10000
DEG = 16
ESZ = 3

NC = 2
NS = 16
NW = NC * NS

C_NODES = 2
ROWS_PER_CHUNK = C_NODES * DEG * ESZ
B_PAD = 10240
PAIR_NODES = 2 * (B_PAD // NW)
SLOW_NODES = 288
FAST_NODES = PAIR_NODES - SLOW_NODES
NBUF = 2
DC = D // 16


def _body(table_hbm, idx_hbm, w_hbm, b_hbm, out_hbm,
          idx_stage, rows0, rows1, wv, bv, out_stage, sem0, sem1):
    rows_bufs = (rows0, rows1)
    sems = (sem0, sem1)
    sid = lax.axis_index("s")
    cid = lax.axis_index("c")

    pltpu.sync_copy(w_hbm, wv)
    pltpu.sync_copy(b_hbm, bv)
    wvec = wv[...] * jnp.float32(1.0 / (9.0 * D))
    bvec = bv[...]

    three = jnp.float32(3.0)

    def pipeline(n_nodes, node_base):
        node_base = pl.multiple_of(node_base, 16)
        n_chunks = n_nodes // C_NODES
        idx_base = pl.multiple_of(
            node_base * (DEG * ESZ) // ROWS_PER_CHUNK, 8)

        pltpu.sync_copy(idx_hbm.at[pl.ds(idx_base, n_chunks)],
                        idx_stage.at[pl.ds(0, n_chunks)])

        def start_gather(g, slot):
            pltpu.make_async_copy(table_hbm.at[idx_stage.at[g]],
                                  rows_bufs[slot], sems[slot]).start()

        def wait_gather(slot):
            pltpu.make_async_copy(table_hbm.at[idx_stage.at[0]],
                                  rows_bufs[slot], sems[slot]).wait()

        for s0 in range(NBUF):
            start_gather(s0, s0)

        def compute_chunk(g, slot):
            rows = rows_bufs[slot]
            for n in range(C_NODES):
                acc = [jnp.zeros((16,), jnp.float32) for _ in range(DC)]
                for e in range(DEG):
                    base = n * DEG * ESZ + e * ESZ
                    vsum = jnp.zeros((16,), jnp.float32)
                    msg = []
                    for dc in range(DC):
                        sl = pl.ds(dc * 16, 16)
                        f0 = rows[base + 0, sl]
                        f1 = rows[base + 1, sl]
                        f2 = rows[base + 2, sl]
                        s = f0 + f1 + f2
                        q = f0 * f0 + f1 * f1 + f2 * f2
                        vsum = vsum + (q * three - s * s)
                        msg.append(f0 * f1)
                    ev = jnp.sum(vsum)
                    evv = jnp.broadcast_to(ev, (16,))
                    z = evv * wvec + bvec
                    att = 1.0 / (1.0 + jnp.exp(-z))
                    for dc in range(DC):
                        acc[dc] = acc[dc] + att * msg[dc]
                row = g * C_NODES + n
                for dc in range(DC):
                    out_stage[row, pl.ds(dc * 16, 16)] = acc[dc]

        def group(i, _):
            g0 = i * NBUF
            for slot in range(NBUF):
                g = g0 + slot
                wait_gather(slot)
                compute_chunk(g, slot)

                @pl.when(g + NBUF < n_chunks)
                def _():
                    start_gather(g + NBUF, slot)
            return _

        lax.fori_loop(0, n_chunks // NBUF, group, None)
        pltpu.sync_copy(out_stage.at[pl.ds(0, n_nodes)],
                        out_hbm.at[pl.ds(node_base, n_nodes)])

    @pl.when(cid == 0)
    def _():
        pipeline(FAST_NODES, sid * PAIR_NODES)

    @pl.when(cid == 1)
    def _():
        pipeline(SLOW_NODES, sid * PAIR_NODES + FAST_NODES)


@jax.jit
def _run(edge_idx, table, w_vec, b_vec):
    mesh = plsc.VectorSubcoreMesh(core_axis_name="c", subcore_axis_name="s")
    f = pl.kernel(
        _body,
        out_type=jax.ShapeDtypeStruct((B_PAD, D), jnp.float32),
        mesh=mesh,
        compiler_params=pltpu.CompilerParams(needs_layout_passes=False),
        scratch_types=[
            pltpu.VMEM((FAST_NODES // C_NODES, ROWS_PER_CHUNK), jnp.int32),
            pltpu.VMEM((ROWS_PER_CHUNK, D), jnp.float32),
            pltpu.VMEM((ROWS_PER_CHUNK, D), jnp.float32),
            pltpu.VMEM((16,), jnp.float32),
            pltpu.VMEM((16,), jnp.float32),
            pltpu.VMEM((FAST_NODES, D), jnp.float32),
            pltpu.SemaphoreType.DMA,
            pltpu.SemaphoreType.DMA,
        ],
    )
    return f(table, edge_idx, w_vec, b_vec)


def kernel(nodes, edge_nodes, table, w_att_w, w_att_b):
    del nodes
    idx = edge_nodes.reshape(B, DEG * ESZ)
    idx = jnp.pad(idx, ((0, B_PAD - B), (0, 0)))
    idx = idx.reshape(B_PAD * DEG * ESZ // ROWS_PER_CHUNK, ROWS_PER_CHUNK)
    w_vec = jnp.full((16,), w_att_w[0, 0], jnp.float32)
    b_vec = jnp.full((16,), w_att_b[0], jnp.float32)
    out = _run(idx, table, w_vec, b_vec)
    return out[:B]


# speedup vs baseline: 1.0428x; 1.0428x over previous
import functools

import jax
import jax.numpy as jnp
from jax import lax
from jax.experimental import pallas as pl
from jax.experimental.pallas import tpu as pltpu
from jax.experimental.pallas import tpu_sc as plsc

N_NODES = 100000
D = 128
B = 10000
DEG = 16
ESZ = 3

NC = 2
NS = 16
NW = NC * NS

C_NODES = 2
ROWS_PER_CHUNK = C_NODES * DEG * ESZ
B_PAD = 10240
PAIR_NODES = 2 * (B_PAD // NW)
SLOW_NODES = 256
FAST_NODES = PAIR_NODES - SLOW_NODES
NBUF = 2
DC = D // 16


def _body(table_hbm, idx_hbm, w_hbm, b_hbm, out_hbm,
          idx_stage, rows0, rows1, wv, bv, out_stage, sem0, sem1):
    rows_bufs = (rows0, rows1)
    sems = (sem0, sem1)
    sid = lax.axis_index("s")
    cid = lax.axis_index("c")

    pltpu.sync_copy(w_hbm, wv)
    pltpu.sync_copy(b_hbm, bv)
    wvec = wv[...] * jnp.float32(1.0 / (9.0 * D))
    bvec = bv[...]

    three = jnp.float32(3.0)

    def pipeline(n_nodes, node_base):
        node_base = pl.multiple_of(node_base, 16)
        n_chunks = n_nodes // C_NODES
        idx_base = pl.multiple_of(
            node_base * (DEG * ESZ) // ROWS_PER_CHUNK, 8)

        pltpu.sync_copy(idx_hbm.at[pl.ds(idx_base, n_chunks)],
                        idx_stage.at[pl.ds(0, n_chunks)])

        def start_gather(g, slot):
            pltpu.make_async_copy(table_hbm.at[idx_stage.at[g]],
                                  rows_bufs[slot], sems[slot]).start()

        def wait_gather(slot):
            pltpu.make_async_copy(table_hbm.at[idx_stage.at[0]],
                                  rows_bufs[slot], sems[slot]).wait()

        for s0 in range(NBUF):
            start_gather(s0, s0)

        def compute_chunk(g, slot):
            rows = rows_bufs[slot]
            for n in range(C_NODES):
                acc = [jnp.zeros((16,), jnp.float32) for _ in range(DC)]
                for e in range(DEG):
                    base = n * DEG * ESZ + e * ESZ
                    vsum = jnp.zeros((16,), jnp.float32)
                    msg = []
                    for dc in range(DC):
                        sl = pl.ds(dc * 16, 16)
                        f0 = rows[base + 0, sl]
                        f1 = rows[base + 1, sl]
                        f2 = rows[base + 2, sl]
                        s = f0 + f1 + f2
                        q = f0 * f0 + f1 * f1 + f2 * f2
                        vsum = vsum + (q * three - s * s)
                        msg.append(f0 * f1)
                    ev = jnp.sum(vsum)
                    evv = jnp.broadcast_to(ev, (16,))
                    z = evv * wvec + bvec
                    att = 1.0 / (1.0 + jnp.exp(-z))
                    for dc in range(DC):
                        acc[dc] = acc[dc] + att * msg[dc]
                row = g * C_NODES + n
                for dc in range(DC):
                    out_stage[row, pl.ds(dc * 16, 16)] = acc[dc]

        def group(i, _):
            g0 = i * NBUF
            for slot in range(NBUF):
                g = g0 + slot
                wait_gather(slot)
                compute_chunk(g, slot)

                @pl.when(g + NBUF < n_chunks)
                def _():
                    start_gather(g + NBUF, slot)
            return _

        lax.fori_loop(0, n_chunks // NBUF, group, None)
        pltpu.sync_copy(out_stage.at[pl.ds(0, n_nodes)],
                        out_hbm.at[pl.ds(node_base, n_nodes)])

    @pl.when(cid == 0)
    def _():
        pipeline(FAST_NODES, sid * PAIR_NODES)

    @pl.when(cid == 1)
    def _():
        pipeline(SLOW_NODES, sid * PAIR_NODES + FAST_NODES)


@jax.jit
def _run(edge_idx, table, w_vec, b_vec):
    mesh = plsc.VectorSubcoreMesh(core_axis_name="c", subcore_axis_name="s")
    f = pl.kernel(
        _body,
        out_type=jax.ShapeDtypeStruct((B_PAD, D), jnp.float32),
        mesh=mesh,
        compiler_params=pltpu.CompilerParams(needs_layout_passes=False),
        scratch_types=[
            pltpu.VMEM((FAST_NODES // C_NODES, ROWS_PER_CHUNK), jnp.int32),
            pltpu.VMEM((ROWS_PER_CHUNK, D), jnp.float32),
            pltpu.VMEM((ROWS_PER_CHUNK, D), jnp.float32),
            pltpu.VMEM((16,), jnp.float32),
            pltpu.VMEM((16,), jnp.float32),
            pltpu.VMEM((FAST_NODES, D), jnp.float32),
            pltpu.SemaphoreType.DMA,
            pltpu.SemaphoreType.DMA,
        ],
    )
    return f(table, edge_idx, w_vec, b_vec)


def kernel(nodes, edge_nodes, table, w_att_w, w_att_b):
    del nodes
    idx = edge_nodes.reshape(B, DEG * ESZ)
    idx = jnp.pad(idx, ((0, B_PAD - B), (0, 0)))
    idx = idx.reshape(B_PAD * DEG * ESZ // ROWS_PER_CHUNK, ROWS_PER_CHUNK)
    w_vec = jnp.full((16,), w_att_w[0, 0], jnp.float32)
    b_vec = jnp.full((16,), w_att_b[0], jnp.float32)
    out = _run(idx, table, w_vec, b_vec)
    return out[:B]
